# baseline (speedup 1.0000x reference)
import functools

import jax
import jax.numpy as jnp
from jax import lax
from jax.experimental import pallas as pl
from jax.experimental.pallas import tpu as pltpu
from jax.experimental.pallas import tpu_sc as plsc

_N = 10000
_E = 320000
_B = 64
_NF = 128
_EF = 16
_H = 300
_HP = 384
_EMB = 1024
_PH = 512

_NS = 16
_NH = _N // 2
_DUMP = 8
_SCN = 3200
_R = 80
_EB = 2000
_NB = 1000

_f32 = jnp.float32
_i32 = jnp.int32


def _mm(a, b):
    return jnp.dot(a, b, preferred_element_type=_f32)


def _node_embed_body(x_ref, w_ref, b_ref, out_ref):
    out_ref[...] = _mm(x_ref[...], w_ref[...]) + b_ref[...]


def _edge_embed_body(ea_ref, w_ref, b_ref, out_ref):
    ea = ea_ref[...]
    for l in range(3):
        out_ref[l] = jnp.maximum(_mm(ea, w_ref[l]) + b_ref[l], 0.0)


def _mlp_body(z_ref, w1_ref, b1_ref, w2_ref, b2_ref, out_ref):
    t = jnp.maximum(_mm(z_ref[...], w1_ref[...]) + b1_ref[...], 0.0)
    out_ref[...] = jnp.maximum(_mm(t, w2_ref[...]) + b2_ref[...], 0.0)


def _readout_body(h_ref, batch_ref, wr_ref, br_ref,
                  wp1_ref, bp1_ref, a1_ref, wp2_ref, bp2_ref, a2_ref,
                  wp3_ref, bp3_ref, out_ref, acc):
    i = pl.program_id(0)

    @pl.when(i == 0)
    def _():
        acc[...] = jnp.zeros_like(acc)

    ids = lax.broadcasted_iota(_i32, (_B, _NB), 0)
    m = (ids == jnp.broadcast_to(batch_ref[0], (_B, _NB))).astype(_f32)
    acc[...] += _mm(m, h_ref[...])

    @pl.when(i == pl.num_programs(0) - 1)
    def _():
        g = jnp.maximum(_mm(acc[...], wr_ref[...]) + br_ref[...], 0.0)
        o = _mm(g, wp1_ref[...]) + bp1_ref[...]
        o = jnp.where(o > 0, o, a1_ref[0, 0] * o)
        o = _mm(o, wp2_ref[...]) + bp2_ref[...]
        o = jnp.where(o > 0, o, a2_ref[0, 0] * o)
        out_ref[...] = _mm(o, wp3_ref[...]) + bp3_ref[...]


def _sc_msg_body(layer, src_hbm, dst_hbm, h_hbm, e_hbm, z_hbm,
                 srcb, dstb, cpk, ceid, e_buf, h_buf,
                 agg, sem_e, sem_h):
    c = lax.axis_index("c")
    s = lax.axis_index("s")
    lane = lax.broadcasted_iota(_i32, (16,), 0)

    for p in range(4):
        rank = (p * 2 + c) * 16 + s
        mybase = rank * _R
        alive = rank < _N // _R
        mybv = jnp.full((16,), mybase, _i32)

        @pl.when(alive)
        def _():
            pltpu.sync_copy(h_hbm.at[pl.ds(mybase, _R)], agg.at[pl.ds(0, _R)])

        def round_body(j, _):
            base = j * _SCN
            cs = pltpu.async_copy(src_hbm.at[pl.ds(base, _SCN)], srcb, sem_e)
            cd = pltpu.async_copy(dst_hbm.at[pl.ds(base, _SCN)], dstb, sem_h)
            cs.wait()
            cd.wait()

            def scan_group(g, cnt):
                sl = pl.ds(g * 16, 16)
                local = dstb[sl] - mybv
                owned = (local >= 0) & (local < _R)
                pos = plsc.cumsum(owned.astype(_i32))
                idx = jnp.full((16,), cnt - 1, _i32) + pos
                eid = jnp.full((16,), layer * _E + base + g * 16, _i32) + lane
                plsc.store_scatter(cpk, [idx], local | (srcb[sl] << 16),
                                   mask=owned)
                plsc.store_scatter(ceid, [idx], eid, mask=owned)
                return cnt + jnp.max(pos)

            cnt = lax.fori_loop(0, _SCN // 16, scan_group, jnp.int32(0))
            kpad = ((cnt + 15) // 16) * 16

            idx = jnp.full((16,), cnt, _i32) + lane
            msk = idx < jnp.full((16,), kpad, _i32)
            plsc.store_scatter(cpk, [idx], _R + (lane & (_DUMP - 1)),
                               mask=msk)
            plsc.store_scatter(ceid, [idx],
                               jnp.full((16,), layer * _E, _i32), mask=msk)

            def chunk(i, _):
                o = pl.ds(i * 16, 16)
                v = cpk[o]
                cp_e = pltpu.async_copy(
                    e_hbm.at[plsc.Indices(ceid[o])], e_buf, sem_e)
                cp_h = pltpu.async_copy(
                    h_hbm.at[plsc.Indices(v >> 16)], h_buf, sem_h)
                dl = v & 0xFFFF
                cp_e.wait()
                cp_h.wait()

                def col(kc, _):
                    kv = jnp.full((16,), kc, _i32)
                    xe = plsc.load_gather(e_buf, [lane, kv])
                    xh = plsc.load_gather(h_buf, [lane, kv])
                    plsc.addupdate_scatter(
                        agg, [dl, kv], jnp.maximum(xe + xh, 0.0))
                    return 0

                lax.fori_loop(0, _HP, col, 0)
                return 0

            lax.fori_loop(0, kpad // 16, chunk, 0)
            return 0

        @pl.when(alive)
        def _():
            lax.fori_loop(0, _E // _SCN, round_body, 0)
            pltpu.sync_copy(agg.at[pl.ds(0, _R)], z_hbm.at[pl.ds(mybase, _R)])


def _sc_msg(layer, src, dst, h, e_all):
    mesh = plsc.VectorSubcoreMesh(core_axis_name="c", subcore_axis_name="s")
    fn = pl.kernel(
        functools.partial(_sc_msg_body, layer),
        out_type=jax.ShapeDtypeStruct((_N, _HP), _f32),
        mesh=mesh,
        compiler_params=pltpu.CompilerParams(needs_layout_passes=False),
        scratch_types=[
            pltpu.VMEM((_SCN,), _i32),
            pltpu.VMEM((_SCN,), _i32),
            pltpu.VMEM((_SCN + 16,), _i32),
            pltpu.VMEM((_SCN + 16,), _i32),
            pltpu.VMEM((16, _HP), _f32),
            pltpu.VMEM((16, _HP), _f32),
            pltpu.VMEM((_R + _DUMP, _HP), _f32),
            pltpu.SemaphoreType.DMA,
            pltpu.SemaphoreType.DMA,
        ],
    )
    return fn(src, dst, h, e_all)


def kernel(x, edge_index, edge_attr, batch, W_node, b_node, W_edge0, b_edge0,
           W1_0, b1_0, W2_0, b2_0, W_edge1, b_edge1, W1_1, b1_1, W2_1, b2_1,
           W_edge2, b_edge2, W1_2, b1_2, W2_2, b2_2, W_read, b_read,
           Wp1, bp1, a1, Wp2, bp2, a2, Wp3, bp3):
    src = edge_index[0]
    dst = edge_index[1]
    batch3 = batch.reshape(_N // _NB, 1, _NB)

    pad_c = lambda w: jnp.pad(w, ((0, 0), (0, _HP - _H)))
    pad_r = lambda w: jnp.pad(w, ((0, _HP - _H), (0, 0)))
    wn = pad_c(W_node)
    bn = jnp.pad(b_node, (0, _HP - _H)).reshape(1, _HP)
    we = jnp.stack([pad_c(W_edge0), pad_c(W_edge1), pad_c(W_edge2)])
    be = jnp.stack([jnp.pad(b_edge0, (0, _HP - _H)),
                    jnp.pad(b_edge1, (0, _HP - _H)),
                    jnp.pad(b_edge2, (0, _HP - _H))]).reshape(3, 1, _HP)
    mlpw = [(pad_r(W1_0), b1_0.reshape(1, 600), pad_c(W2_0),
             jnp.pad(b2_0, (0, _HP - _H)).reshape(1, _HP)),
            (pad_r(W1_1), b1_1.reshape(1, 600), pad_c(W2_1),
             jnp.pad(b2_1, (0, _HP - _H)).reshape(1, _HP)),
            (pad_r(W1_2), b1_2.reshape(1, 600), pad_c(W2_2),
             jnp.pad(b2_2, (0, _HP - _H)).reshape(1, _HP))]
    wr = pad_r(W_read)
    brr = b_read.reshape(1, _EMB)
    bp1r = bp1.reshape(1, _PH)
    bp2r = bp2.reshape(1, _PH)
    a1r = a1.reshape(1, 1)
    a2r = a2.reshape(1, 1)
    bp3r = bp3.reshape(1, 1)

    full = lambda shape: pl.BlockSpec(shape, lambda i: (0,) * len(shape))

    h = pl.pallas_call(
        _node_embed_body,
        grid=(_N // _NB,),
        in_specs=[pl.BlockSpec((_NB, _NF), lambda i: (i, 0)),
                  full((_NF, _HP)), full((1, _HP))],
        out_specs=pl.BlockSpec((_NB, _HP), lambda i: (i, 0)),
        out_shape=jax.ShapeDtypeStruct((_N, _HP), _f32),
    )(x, wn, bn)

    e4 = pl.pallas_call(
        _edge_embed_body,
        grid=(_E // _EB,),
        in_specs=[pl.BlockSpec((_EB, _EF), lambda i: (i, 0)),
                  full((3, _EF, _HP)), full((3, 1, _HP))],
        out_specs=pl.BlockSpec((3, _EB, _HP), lambda i: (0, i, 0)),
        out_shape=jax.ShapeDtypeStruct((3, _E, _HP), _f32),
    )(edge_attr, we, be)
    e_all = e4.reshape(3 * _E, _HP)

    for l in range(3):
        z = _sc_msg(l, src, dst, h, e_all)
        w1, b1, w2, b2 = mlpw[l]
        h = pl.pallas_call(
            _mlp_body,
            grid=(_N // _NB,),
            in_specs=[pl.BlockSpec((_NB, _HP), lambda i: (i, 0)),
                      full((_HP, 600)), full((1, 600)),
                      full((600, _HP)), full((1, _HP))],
            out_specs=pl.BlockSpec((_NB, _HP), lambda i: (i, 0)),
            out_shape=jax.ShapeDtypeStruct((_N, _HP), _f32),
        )(z, w1, b1, w2, b2)

    out = pl.pallas_call(
        _readout_body,
        grid=(_N // _NB,),
        in_specs=[pl.BlockSpec((_NB, _HP), lambda i: (i, 0)),
                  pl.BlockSpec((1, 1, _NB), lambda i: (i, 0, 0)),
                  full((_HP, _EMB)), full((1, _EMB)),
                  full((_EMB, _PH)), full((1, _PH)), full((1, 1)),
                  full((_PH, _PH)), full((1, _PH)), full((1, 1)),
                  full((_PH, 1)), full((1, 1))],
        out_specs=pl.BlockSpec((_B, 1), lambda i: (0, 0)),
        out_shape=jax.ShapeDtypeStruct((_B, 1), _f32),
        scratch_shapes=[pltpu.VMEM((_B, _HP), _f32)],
        compiler_params=pltpu.CompilerParams(
            dimension_semantics=("arbitrary",)),
    )(h, batch3, wr, brr, Wp1, bp1r, a1r, Wp2, bp2r, a2r, Wp3, bp3r)
    return out


# speedup vs baseline: 1.0413x; 1.0413x over previous
import functools

import jax
import jax.numpy as jnp
from jax import lax
from jax.experimental import pallas as pl
from jax.experimental.pallas import tpu as pltpu
from jax.experimental.pallas import tpu_sc as plsc

_N = 10000
_E = 320000
_B = 64
_NF = 128
_EF = 16
_H = 300
_HP = 384
_EMB = 1024
_PH = 512

_NS = 16
_NH = _N // 2
_DUMP = 8
_SCN = 4000
_R = 80
_EB = 2000
_NB = 1000

_f32 = jnp.float32
_i32 = jnp.int32


def _mm(a, b):
    return jnp.dot(a, b, preferred_element_type=_f32)


def _node_embed_body(x_ref, w_ref, b_ref, out_ref):
    out_ref[...] = _mm(x_ref[...], w_ref[...]) + b_ref[...]


def _edge_embed_body(ea_ref, w_ref, b_ref, out_ref):
    ea = ea_ref[...]
    for l in range(3):
        out_ref[l] = jnp.maximum(_mm(ea, w_ref[l]) + b_ref[l], 0.0)


def _mlp_body(z_ref, w1_ref, b1_ref, w2_ref, b2_ref, out_ref):
    t = jnp.maximum(_mm(z_ref[...], w1_ref[...]) + b1_ref[...], 0.0)
    out_ref[...] = jnp.maximum(_mm(t, w2_ref[...]) + b2_ref[...], 0.0)


def _readout_body(h_ref, batch_ref, wr_ref, br_ref,
                  wp1_ref, bp1_ref, a1_ref, wp2_ref, bp2_ref, a2_ref,
                  wp3_ref, bp3_ref, out_ref, acc):
    i = pl.program_id(0)

    @pl.when(i == 0)
    def _():
        acc[...] = jnp.zeros_like(acc)

    ids = lax.broadcasted_iota(_i32, (_B, _NB), 0)
    m = (ids == jnp.broadcast_to(batch_ref[0], (_B, _NB))).astype(_f32)
    acc[...] += _mm(m, h_ref[...])

    @pl.when(i == pl.num_programs(0) - 1)
    def _():
        g = jnp.maximum(_mm(acc[...], wr_ref[...]) + br_ref[...], 0.0)
        o = _mm(g, wp1_ref[...]) + bp1_ref[...]
        o = jnp.where(o > 0, o, a1_ref[0, 0] * o)
        o = _mm(o, wp2_ref[...]) + bp2_ref[...]
        o = jnp.where(o > 0, o, a2_ref[0, 0] * o)
        out_ref[...] = _mm(o, wp3_ref[...]) + bp3_ref[...]


def _sc_msg_body(layer, src_hbm, dst_hbm, h_hbm, e_hbm, z_hbm,
                 srcb, dstb, cpk, ceid, e_buf, h_buf,
                 agg, sem_e, sem_h):
    c = lax.axis_index("c")
    s = lax.axis_index("s")
    lane = lax.broadcasted_iota(_i32, (16,), 0)

    for p in range(4):
        rank = (p * 2 + c) * 16 + s
        mybase = rank * _R
        alive = rank < _N // _R
        mybv = jnp.full((16,), mybase, _i32)

        @pl.when(alive)
        def _():
            pltpu.sync_copy(h_hbm.at[pl.ds(mybase, _R)], agg.at[pl.ds(0, _R)])

        def round_body(j, _):
            base = j * _SCN
            cs = pltpu.async_copy(src_hbm.at[pl.ds(base, _SCN)], srcb, sem_e)
            cd = pltpu.async_copy(dst_hbm.at[pl.ds(base, _SCN)], dstb, sem_h)
            cs.wait()
            cd.wait()

            def scan_group(g, cnt):
                sl = pl.ds(g * 16, 16)
                local = dstb[sl] - mybv
                owned = (local >= 0) & (local < _R)
                pos = plsc.cumsum(owned.astype(_i32))
                idx = jnp.full((16,), cnt - 1, _i32) + pos
                eid = jnp.full((16,), layer * _E + base + g * 16, _i32) + lane
                plsc.store_scatter(cpk, [idx], local | (srcb[sl] << 16),
                                   mask=owned)
                plsc.store_scatter(ceid, [idx], eid, mask=owned)
                return cnt + jnp.max(pos)

            cnt = lax.fori_loop(0, _SCN // 16, scan_group, jnp.int32(0))
            kpad = ((cnt + 15) // 16) * 16

            idx = jnp.full((16,), cnt, _i32) + lane
            msk = idx < jnp.full((16,), kpad, _i32)
            plsc.store_scatter(cpk, [idx], _R + (lane & (_DUMP - 1)),
                               mask=msk)
            plsc.store_scatter(ceid, [idx],
                               jnp.full((16,), layer * _E, _i32), mask=msk)

            def chunk(i, _):
                o = pl.ds(i * 16, 16)
                v = cpk[o]
                cp_e = pltpu.async_copy(
                    e_hbm.at[plsc.Indices(ceid[o])], e_buf, sem_e)
                cp_h = pltpu.async_copy(
                    h_hbm.at[plsc.Indices(v >> 16)], h_buf, sem_h)
                dl = v & 0xFFFF
                cp_e.wait()
                cp_h.wait()

                def col(kc, _):
                    kv = jnp.full((16,), kc, _i32)
                    xe = plsc.load_gather(e_buf, [lane, kv])
                    xh = plsc.load_gather(h_buf, [lane, kv])
                    plsc.addupdate_scatter(
                        agg, [dl, kv], jnp.maximum(xe + xh, 0.0))
                    return 0

                lax.fori_loop(0, _HP, col, 0)
                return 0

            lax.fori_loop(0, kpad // 16, chunk, 0)
            return 0

        @pl.when(alive)
        def _():
            lax.fori_loop(0, _E // _SCN, round_body, 0)
            pltpu.sync_copy(agg.at[pl.ds(0, _R)], z_hbm.at[pl.ds(mybase, _R)])


def _sc_msg(layer, src, dst, h, e_all):
    mesh = plsc.VectorSubcoreMesh(core_axis_name="c", subcore_axis_name="s")
    fn = pl.kernel(
        functools.partial(_sc_msg_body, layer),
        out_type=jax.ShapeDtypeStruct((_N, _HP), _f32),
        mesh=mesh,
        compiler_params=pltpu.CompilerParams(needs_layout_passes=False),
        scratch_types=[
            pltpu.VMEM((_SCN,), _i32),
            pltpu.VMEM((_SCN,), _i32),
            pltpu.VMEM((_SCN + 16,), _i32),
            pltpu.VMEM((_SCN + 16,), _i32),
            pltpu.VMEM((16, _HP), _f32),
            pltpu.VMEM((16, _HP), _f32),
            pltpu.VMEM((_R + _DUMP, _HP), _f32),
            pltpu.SemaphoreType.DMA,
            pltpu.SemaphoreType.DMA,
        ],
    )
    return fn(src, dst, h, e_all)


def kernel(x, edge_index, edge_attr, batch, W_node, b_node, W_edge0, b_edge0,
           W1_0, b1_0, W2_0, b2_0, W_edge1, b_edge1, W1_1, b1_1, W2_1, b2_1,
           W_edge2, b_edge2, W1_2, b1_2, W2_2, b2_2, W_read, b_read,
           Wp1, bp1, a1, Wp2, bp2, a2, Wp3, bp3):
    src = edge_index[0]
    dst = edge_index[1]
    batch3 = batch.reshape(_N // _NB, 1, _NB)

    pad_c = lambda w: jnp.pad(w, ((0, 0), (0, _HP - _H)))
    pad_r = lambda w: jnp.pad(w, ((0, _HP - _H), (0, 0)))
    wn = pad_c(W_node)
    bn = jnp.pad(b_node, (0, _HP - _H)).reshape(1, _HP)
    we = jnp.stack([pad_c(W_edge0), pad_c(W_edge1), pad_c(W_edge2)])
    be = jnp.stack([jnp.pad(b_edge0, (0, _HP - _H)),
                    jnp.pad(b_edge1, (0, _HP - _H)),
                    jnp.pad(b_edge2, (0, _HP - _H))]).reshape(3, 1, _HP)
    mlpw = [(pad_r(W1_0), b1_0.reshape(1, 600), pad_c(W2_0),
             jnp.pad(b2_0, (0, _HP - _H)).reshape(1, _HP)),
            (pad_r(W1_1), b1_1.reshape(1, 600), pad_c(W2_1),
             jnp.pad(b2_1, (0, _HP - _H)).reshape(1, _HP)),
            (pad_r(W1_2), b1_2.reshape(1, 600), pad_c(W2_2),
             jnp.pad(b2_2, (0, _HP - _H)).reshape(1, _HP))]
    wr = pad_r(W_read)
    brr = b_read.reshape(1, _EMB)
    bp1r = bp1.reshape(1, _PH)
    bp2r = bp2.reshape(1, _PH)
    a1r = a1.reshape(1, 1)
    a2r = a2.reshape(1, 1)
    bp3r = bp3.reshape(1, 1)

    full = lambda shape: pl.BlockSpec(shape, lambda i: (0,) * len(shape))

    h = pl.pallas_call(
        _node_embed_body,
        grid=(_N // _NB,),
        in_specs=[pl.BlockSpec((_NB, _NF), lambda i: (i, 0)),
                  full((_NF, _HP)), full((1, _HP))],
        out_specs=pl.BlockSpec((_NB, _HP), lambda i: (i, 0)),
        out_shape=jax.ShapeDtypeStruct((_N, _HP), _f32),
    )(x, wn, bn)

    e4 = pl.pallas_call(
        _edge_embed_body,
        grid=(_E // _EB,),
        in_specs=[pl.BlockSpec((_EB, _EF), lambda i: (i, 0)),
                  full((3, _EF, _HP)), full((3, 1, _HP))],
        out_specs=pl.BlockSpec((3, _EB, _HP), lambda i: (0, i, 0)),
        out_shape=jax.ShapeDtypeStruct((3, _E, _HP), _f32),
    )(edge_attr, we, be)
    e_all = e4.reshape(3 * _E, _HP)

    for l in range(3):
        z = _sc_msg(l, src, dst, h, e_all)
        w1, b1, w2, b2 = mlpw[l]
        h = pl.pallas_call(
            _mlp_body,
            grid=(_N // _NB,),
            in_specs=[pl.BlockSpec((_NB, _HP), lambda i: (i, 0)),
                      full((_HP, 600)), full((1, 600)),
                      full((600, _HP)), full((1, _HP))],
            out_specs=pl.BlockSpec((_NB, _HP), lambda i: (i, 0)),
            out_shape=jax.ShapeDtypeStruct((_N, _HP), _f32),
        )(z, w1, b1, w2, b2)

    out = pl.pallas_call(
        _readout_body,
        grid=(_N // _NB,),
        in_specs=[pl.BlockSpec((_NB, _HP), lambda i: (i, 0)),
                  pl.BlockSpec((1, 1, _NB), lambda i: (i, 0, 0)),
                  full((_HP, _EMB)), full((1, _EMB)),
                  full((_EMB, _PH)), full((1, _PH)), full((1, 1)),
                  full((_PH, _PH)), full((1, _PH)), full((1, 1)),
                  full((_PH, 1)), full((1, 1))],
        out_specs=pl.BlockSpec((_B, 1), lambda i: (0, 0)),
        out_shape=jax.ShapeDtypeStruct((_B, 1), _f32),
        scratch_shapes=[pltpu.VMEM((_B, _HP), _f32)],
        compiler_params=pltpu.CompilerParams(
            dimension_semantics=("arbitrary",)),
    )(h, batch3, wr, brr, Wp1, bp1r, a1r, Wp2, bp2r, a2r, Wp3, bp3r)
    return out


# speedup vs baseline: 1.0414x; 1.0002x over previous
import functools

import jax
import jax.numpy as jnp
from jax import lax
from jax.experimental import pallas as pl
from jax.experimental.pallas import tpu as pltpu
from jax.experimental.pallas import tpu_sc as plsc

_N = 10000
_E = 320000
_B = 64
_NF = 128
_EF = 16
_H = 300
_HP = 384
_EMB = 1024
_PH = 512

_NS = 16
_NH = _N // 2
_DUMP = 8
_SCN = 4000
_R = 80
_HPB = _HP + 8
_EB = 2000
_NB = 1000

_f32 = jnp.float32
_i32 = jnp.int32


def _mm(a, b):
    return jnp.dot(a, b, preferred_element_type=_f32)


def _node_embed_body(x_ref, w_ref, b_ref, out_ref):
    out_ref[...] = _mm(x_ref[...], w_ref[...]) + b_ref[...]


def _edge_embed_body(ea_ref, w_ref, b_ref, out_ref):
    ea = ea_ref[...]
    for l in range(3):
        out_ref[l] = jnp.maximum(_mm(ea, w_ref[l]) + b_ref[l], 0.0)


def _mlp_body(z_ref, w1_ref, b1_ref, w2_ref, b2_ref, out_ref):
    t = jnp.maximum(_mm(z_ref[...], w1_ref[...]) + b1_ref[...], 0.0)
    out_ref[...] = jnp.maximum(_mm(t, w2_ref[...]) + b2_ref[...], 0.0)


def _readout_body(h_ref, batch_ref, wr_ref, br_ref,
                  wp1_ref, bp1_ref, a1_ref, wp2_ref, bp2_ref, a2_ref,
                  wp3_ref, bp3_ref, out_ref, acc):
    i = pl.program_id(0)

    @pl.when(i == 0)
    def _():
        acc[...] = jnp.zeros_like(acc)

    ids = lax.broadcasted_iota(_i32, (_B, _NB), 0)
    m = (ids == jnp.broadcast_to(batch_ref[0], (_B, _NB))).astype(_f32)
    acc[...] += _mm(m, h_ref[...])

    @pl.when(i == pl.num_programs(0) - 1)
    def _():
        g = jnp.maximum(_mm(acc[...], wr_ref[...]) + br_ref[...], 0.0)
        o = _mm(g, wp1_ref[...]) + bp1_ref[...]
        o = jnp.where(o > 0, o, a1_ref[0, 0] * o)
        o = _mm(o, wp2_ref[...]) + bp2_ref[...]
        o = jnp.where(o > 0, o, a2_ref[0, 0] * o)
        out_ref[...] = _mm(o, wp3_ref[...]) + bp3_ref[...]


def _sc_msg_body(layer, src_hbm, dst_hbm, h_hbm, e_hbm, z_hbm,
                 srcb, dstb, cpk, ceid, e_buf, h_buf,
                 agg, sem_e, sem_h):
    c = lax.axis_index("c")
    s = lax.axis_index("s")
    lane = lax.broadcasted_iota(_i32, (16,), 0)

    for p in range(4):
        rank = (p * 2 + c) * 16 + s
        mybase = rank * _R
        alive = rank < _N // _R
        mybv = jnp.full((16,), mybase, _i32)

        @pl.when(alive)
        def _():
            pltpu.sync_copy(h_hbm.at[pl.ds(mybase, _R)],
                            agg.at[pl.ds(0, _R), pl.ds(0, _HP)])

        def round_body(j, _):
            base = j * _SCN
            cs = pltpu.async_copy(src_hbm.at[pl.ds(base, _SCN)], srcb, sem_e)
            cd = pltpu.async_copy(dst_hbm.at[pl.ds(base, _SCN)], dstb, sem_h)
            cs.wait()
            cd.wait()

            def scan_group(g, cnt):
                sl = pl.ds(g * 16, 16)
                local = dstb[sl] - mybv
                owned = (local >= 0) & (local < _R)
                pos = plsc.cumsum(owned.astype(_i32))
                idx = jnp.full((16,), cnt - 1, _i32) + pos
                eid = jnp.full((16,), layer * _E + base + g * 16, _i32) + lane
                plsc.store_scatter(cpk, [idx], local | (srcb[sl] << 16),
                                   mask=owned)
                plsc.store_scatter(ceid, [idx], eid, mask=owned)
                return cnt + jnp.max(pos)

            cnt = lax.fori_loop(0, _SCN // 16, scan_group, jnp.int32(0))
            kpad = ((cnt + 15) // 16) * 16

            idx = jnp.full((16,), cnt, _i32) + lane
            msk = idx < jnp.full((16,), kpad, _i32)
            plsc.store_scatter(cpk, [idx], _R + (lane & (_DUMP - 1)),
                               mask=msk)
            plsc.store_scatter(ceid, [idx],
                               jnp.full((16,), layer * _E, _i32), mask=msk)

            def chunk(i, _):
                o = pl.ds(i * 16, 16)
                v = cpk[o]
                cp_e = pltpu.async_copy(
                    e_hbm.at[plsc.Indices(ceid[o])],
                    e_buf.at[pl.ds(0, 16), pl.ds(0, _HP)], sem_e)
                cp_h = pltpu.async_copy(
                    h_hbm.at[plsc.Indices(v >> 16)],
                    h_buf.at[pl.ds(0, 16), pl.ds(0, _HP)], sem_h)
                dl = v & 0xFFFF
                cp_e.wait()
                cp_h.wait()

                def col(kc, _):
                    kv = jnp.full((16,), kc, _i32)
                    xe = plsc.load_gather(e_buf, [lane, kv])
                    xh = plsc.load_gather(h_buf, [lane, kv])
                    plsc.addupdate_scatter(
                        agg, [dl, kv], jnp.maximum(xe + xh, 0.0))
                    return 0

                lax.fori_loop(0, _HP, col, 0)
                return 0

            lax.fori_loop(0, kpad // 16, chunk, 0)
            return 0

        @pl.when(alive)
        def _():
            lax.fori_loop(0, _E // _SCN, round_body, 0)
            pltpu.sync_copy(agg.at[pl.ds(0, _R), pl.ds(0, _HP)],
                            z_hbm.at[pl.ds(mybase, _R)])


def _sc_msg(layer, src, dst, h, e_all):
    mesh = plsc.VectorSubcoreMesh(core_axis_name="c", subcore_axis_name="s")
    fn = pl.kernel(
        functools.partial(_sc_msg_body, layer),
        out_type=jax.ShapeDtypeStruct((_N, _HP), _f32),
        mesh=mesh,
        compiler_params=pltpu.CompilerParams(needs_layout_passes=False),
        scratch_types=[
            pltpu.VMEM((_SCN,), _i32),
            pltpu.VMEM((_SCN,), _i32),
            pltpu.VMEM((_SCN + 16,), _i32),
            pltpu.VMEM((_SCN + 16,), _i32),
            pltpu.VMEM((16, _HPB), _f32),
            pltpu.VMEM((16, _HPB), _f32),
            pltpu.VMEM((_R + _DUMP, _HPB), _f32),
            pltpu.SemaphoreType.DMA,
            pltpu.SemaphoreType.DMA,
        ],
    )
    return fn(src, dst, h, e_all)


def kernel(x, edge_index, edge_attr, batch, W_node, b_node, W_edge0, b_edge0,
           W1_0, b1_0, W2_0, b2_0, W_edge1, b_edge1, W1_1, b1_1, W2_1, b2_1,
           W_edge2, b_edge2, W1_2, b1_2, W2_2, b2_2, W_read, b_read,
           Wp1, bp1, a1, Wp2, bp2, a2, Wp3, bp3):
    src = edge_index[0]
    dst = edge_index[1]
    batch3 = batch.reshape(_N // _NB, 1, _NB)

    pad_c = lambda w: jnp.pad(w, ((0, 0), (0, _HP - _H)))
    pad_r = lambda w: jnp.pad(w, ((0, _HP - _H), (0, 0)))
    wn = pad_c(W_node)
    bn = jnp.pad(b_node, (0, _HP - _H)).reshape(1, _HP)
    we = jnp.stack([pad_c(W_edge0), pad_c(W_edge1), pad_c(W_edge2)])
    be = jnp.stack([jnp.pad(b_edge0, (0, _HP - _H)),
                    jnp.pad(b_edge1, (0, _HP - _H)),
                    jnp.pad(b_edge2, (0, _HP - _H))]).reshape(3, 1, _HP)
    mlpw = [(pad_r(W1_0), b1_0.reshape(1, 600), pad_c(W2_0),
             jnp.pad(b2_0, (0, _HP - _H)).reshape(1, _HP)),
            (pad_r(W1_1), b1_1.reshape(1, 600), pad_c(W2_1),
             jnp.pad(b2_1, (0, _HP - _H)).reshape(1, _HP)),
            (pad_r(W1_2), b1_2.reshape(1, 600), pad_c(W2_2),
             jnp.pad(b2_2, (0, _HP - _H)).reshape(1, _HP))]
    wr = pad_r(W_read)
    brr = b_read.reshape(1, _EMB)
    bp1r = bp1.reshape(1, _PH)
    bp2r = bp2.reshape(1, _PH)
    a1r = a1.reshape(1, 1)
    a2r = a2.reshape(1, 1)
    bp3r = bp3.reshape(1, 1)

    full = lambda shape: pl.BlockSpec(shape, lambda i: (0,) * len(shape))

    h = pl.pallas_call(
        _node_embed_body,
        grid=(_N // _NB,),
        in_specs=[pl.BlockSpec((_NB, _NF), lambda i: (i, 0)),
                  full((_NF, _HP)), full((1, _HP))],
        out_specs=pl.BlockSpec((_NB, _HP), lambda i: (i, 0)),
        out_shape=jax.ShapeDtypeStruct((_N, _HP), _f32),
    )(x, wn, bn)

    e4 = pl.pallas_call(
        _edge_embed_body,
        grid=(_E // _EB,),
        in_specs=[pl.BlockSpec((_EB, _EF), lambda i: (i, 0)),
                  full((3, _EF, _HP)), full((3, 1, _HP))],
        out_specs=pl.BlockSpec((3, _EB, _HP), lambda i: (0, i, 0)),
        out_shape=jax.ShapeDtypeStruct((3, _E, _HP), _f32),
    )(edge_attr, we, be)
    e_all = e4.reshape(3 * _E, _HP)

    for l in range(3):
        z = _sc_msg(l, src, dst, h, e_all)
        w1, b1, w2, b2 = mlpw[l]
        h = pl.pallas_call(
            _mlp_body,
            grid=(_N // _NB,),
            in_specs=[pl.BlockSpec((_NB, _HP), lambda i: (i, 0)),
                      full((_HP, 600)), full((1, 600)),
                      full((600, _HP)), full((1, _HP))],
            out_specs=pl.BlockSpec((_NB, _HP), lambda i: (i, 0)),
            out_shape=jax.ShapeDtypeStruct((_N, _HP), _f32),
        )(z, w1, b1, w2, b2)

    out = pl.pallas_call(
        _readout_body,
        grid=(_N // _NB,),
        in_specs=[pl.BlockSpec((_NB, _HP), lambda i: (i, 0)),
                  pl.BlockSpec((1, 1, _NB), lambda i: (i, 0, 0)),
                  full((_HP, _EMB)), full((1, _EMB)),
                  full((_EMB, _PH)), full((1, _PH)), full((1, 1)),
                  full((_PH, _PH)), full((1, _PH)), full((1, 1)),
                  full((_PH, 1)), full((1, 1))],
        out_specs=pl.BlockSpec((_B, 1), lambda i: (0, 0)),
        out_shape=jax.ShapeDtypeStruct((_B, 1), _f32),
        scratch_shapes=[pltpu.VMEM((_B, _HP), _f32)],
        compiler_params=pltpu.CompilerParams(
            dimension_semantics=("arbitrary",)),
    )(h, batch3, wr, brr, Wp1, bp1r, a1r, Wp2, bp2r, a2r, Wp3, bp3r)
    return out
